# baseline (speedup 1.0000x reference)
import jax
import jax.numpy as jnp
from jax import lax
from jax.experimental import pallas as pl
from jax.experimental.pallas import tpu as pltpu
from jax.experimental.pallas import tpu_sc as plsc

B = 16384
EMB = 64
NC, NS = 2, 16
NW = NC * NS
BPW = B // NW
HW = 256
CH = 128
NCHT = BPW // CH
CB = 2048


def _gather_body(user_p, movie_p, ids3, out, idx_v, pbuf, sem):
    wid = lax.axis_index("s") * NC + lax.axis_index("c")
    base = wid * BPW
    for t in range(3):
        pltpu.sync_copy(ids3.at[t, pl.ds(wid * NCHT, NCHT)],
                        idx_v.at[pl.ds(t * NCHT, NCHT)])

    for t, table in ((0, user_p), (1, movie_p), (2, movie_p)):
        for hf in range(BPW // HW):
            for ch in range(hf * HW // CH, (hf + 1) * HW // CH):
                def _group(g, _, t=t, table=table, ch=ch, hf=hf):
                    vec = idx_v[t * NCHT + ch, pl.ds(g * 16, 16)]
                    j0 = (ch * CH - hf * HW) + g * 16
                    for u in range(16):
                        quad = lax.shift_right_logical(vec[u], 2)
                        pltpu.async_copy(table.at[pl.ds(quad, 1)],
                                         pbuf.at[pl.ds(j0 + u, 1)], sem)
                    return _

                lax.fori_loop(0, CH // 16, _group, None)
            pltpu.make_async_copy(out.at[t, pl.ds(0, HW)], pbuf, sem).wait()
            pltpu.sync_copy(pbuf, out.at[t, pl.ds(base + hf * HW, HW)])


def _sc_gather(user_p, movie_p, ids3):
    mesh = plsc.VectorSubcoreMesh(core_axis_name="c", subcore_axis_name="s")
    return pl.kernel(
        _gather_body,
        mesh=mesh,
        out_type=jax.ShapeDtypeStruct((3, B, 2 * EMB), jnp.float32),
        scratch_types=[
            pltpu.VMEM((3 * NCHT, CH), jnp.int32),
            pltpu.VMEM((HW, 2 * EMB), jnp.float32),
            pltpu.SemaphoreType.DMA,
        ],
    )(user_p, movie_p, ids3)


def _mlp_body(emb_ref, par_ref, w1_ref, b1_ref, w2_ref, b2_ref,
              ou_ref, op_ref, on_ref):
    t = pl.program_id(1)
    u = lax.bitcast_convert_type(emb_ref[0], jnp.uint32)
    q1 = par_ref[0][:, 0:1] > 0
    q0 = par_ref[0][:, 1:2] > 0
    a = jnp.where(q1, u[:, EMB:], u[:, :EMB])
    bsel = jnp.where(q0, a[:, EMB // 2:], a[:, :EMB // 2])
    lo = lax.bitcast_convert_type(bsel << 16, jnp.float32)
    hi = lax.bitcast_convert_type(
        bsel & jnp.uint32(0xFFFF0000), jnp.float32)
    e = jnp.concatenate([lo, hi], axis=1)
    h = jnp.dot(e, w1_ref[0], preferred_element_type=jnp.float32) + b1_ref[0]
    h = jnp.maximum(h, 0.0)
    o = (jnp.dot(h, w2_ref[0], preferred_element_type=jnp.float32)
         + b2_ref[0])

    @pl.when(t == 0)
    def _():
        ou_ref[...] = o

    @pl.when(t == 1)
    def _():
        op_ref[...] = o

    @pl.when(t == 2)
    def _():
        on_ref[...] = o


def _tc_mlp(emb3, par3, w1s, b1s, w2s, b2s):
    ovec = jax.ShapeDtypeStruct((B, 32), jnp.float32)
    return pl.pallas_call(
        _mlp_body,
        grid=(B // CB, 3),
        in_specs=[
            pl.BlockSpec((1, CB, 2 * EMB), lambda i, t: (t, i, 0)),
            pl.BlockSpec((1, CB, 2), lambda i, t: (t, i, 0)),
            pl.BlockSpec((1, EMB, 64), lambda i, t: (jnp.minimum(t, 1), 0, 0)),
            pl.BlockSpec((1, 1, 64), lambda i, t: (jnp.minimum(t, 1), 0, 0)),
            pl.BlockSpec((1, 64, 32), lambda i, t: (jnp.minimum(t, 1), 0, 0)),
            pl.BlockSpec((1, 1, 32), lambda i, t: (jnp.minimum(t, 1), 0, 0)),
        ],
        out_specs=[
            pl.BlockSpec((CB, 32), lambda i, t: (i, 0)),
            pl.BlockSpec((CB, 32), lambda i, t: (i, 0)),
            pl.BlockSpec((CB, 32), lambda i, t: (i, 0)),
        ],
        out_shape=[ovec, ovec, ovec],
    )(emb3, par3, w1s, b1s, w2s, b2s)


def _pack(table):
    t16 = table.astype(jnp.bfloat16)
    u = lax.bitcast_convert_type(t16.reshape(1000000, EMB // 2, 2),
                                 jnp.uint32)
    return lax.bitcast_convert_type(u.reshape(250000, 2 * EMB), jnp.float32)


def kernel(user_ids, pos_movie_ids, neg_movie_ids, user_table, movie_table,
           uW1, ub1, uW2, ub2, mW1, mb1, mW2, mb2):
    ids = jnp.stack([user_ids, pos_movie_ids, neg_movie_ids]).astype(jnp.int32)
    ids3 = ids.reshape(3, B // CH, CH)
    par3 = jnp.stack([(ids >> 1) & 1, ids & 1],
                     axis=-1).astype(jnp.float32)
    user_p = _pack(user_table)
    movie_p = _pack(movie_table)
    emb3 = _sc_gather(user_p, movie_p, ids3)
    perm = jnp.arange(EMB).reshape(EMB // 2, 2).T.reshape(EMB)
    w1s = jnp.stack([uW1, mW1])[:, perm, :]
    b1s = jnp.stack([ub1, mb1]).reshape(2, 1, 64)
    w2s = jnp.stack([uW2, mW2])
    b2s = jnp.stack([ub2, mb2]).reshape(2, 1, 32)
    return _tc_mlp(emb3, par3, w1s, b1s, w2s, b2s)


# speedup vs baseline: 6.3996x; 6.3996x over previous
import jax
import jax.numpy as jnp
from jax import lax
from jax.experimental import pallas as pl
from jax.experimental.pallas import tpu as pltpu
from jax.experimental.pallas import tpu_sc as plsc

B = 16384
EMB = 64
SUB = 8
NC, NS = 2, 16
NW = NC * NS
BPW = B // NW
CH = 128
NCHT = BPW // CH
CB = 2048


def _gather_body(user_t3, movie_t3, ids3, out, idx_v, rows_v, sem):
    wid = lax.axis_index("s") * NC + lax.axis_index("c")
    base = wid * BPW
    for t in range(3):
        pltpu.sync_copy(ids3.at[t, pl.ds(wid * NCHT, NCHT)],
                        idx_v.at[pl.ds(t * NCHT, NCHT)])

    for t, table in ((0, user_t3), (1, movie_t3), (2, movie_t3)):
        for ch in range(NCHT):
            def _group(g, _, t=t, table=table, ch=ch):
                vec = idx_v[t * NCHT + ch, pl.ds(g * 16, 16)]
                j0 = ch * CH + g * 16
                for u in range(16):
                    rid = vec[u]
                    slab = lax.shift_right_logical(rid, 3)
                    sub = lax.bitwise_and(rid, 7)
                    pltpu.async_copy(table.at[pl.ds(slab, 1), sub],
                                     rows_v.at[pl.ds(j0 + u, 1)], sem)
                return _

            lax.fori_loop(0, CH // 16, _group, None)
        pltpu.make_async_copy(out.at[t, pl.ds(base, BPW)], rows_v, sem).wait()
        pltpu.sync_copy(rows_v, out.at[t, pl.ds(base, BPW)])


def _sc_gather(user_t3, movie_t3, ids3):
    mesh = plsc.VectorSubcoreMesh(core_axis_name="c", subcore_axis_name="s")
    return pl.kernel(
        _gather_body,
        mesh=mesh,
        out_type=jax.ShapeDtypeStruct((3, B, EMB), jnp.float32),
        scratch_types=[
            pltpu.VMEM((3 * NCHT, CH), jnp.int32),
            pltpu.VMEM((BPW, EMB), jnp.float32),
            pltpu.SemaphoreType.DMA,
        ],
    )(user_t3, movie_t3, ids3)


def _mlp_body(emb_ref, w1_ref, b1_ref, w2_ref, b2_ref, ou_ref, op_ref, on_ref):
    t = pl.program_id(1)
    e = emb_ref[0]
    h = jnp.dot(e, w1_ref[0], preferred_element_type=jnp.float32) + b1_ref[0]
    h = jnp.maximum(h, 0.0)
    o = (jnp.dot(h, w2_ref[0], preferred_element_type=jnp.float32)
         + b2_ref[0])

    @pl.when(t == 0)
    def _():
        ou_ref[...] = o

    @pl.when(t == 1)
    def _():
        op_ref[...] = o

    @pl.when(t == 2)
    def _():
        on_ref[...] = o


def _tc_mlp(emb3, w1s, b1s, w2s, b2s):
    ovec = jax.ShapeDtypeStruct((B, 32), jnp.float32)
    return pl.pallas_call(
        _mlp_body,
        grid=(B // CB, 3),
        in_specs=[
            pl.BlockSpec((1, CB, EMB), lambda i, t: (t, i, 0)),
            pl.BlockSpec((1, EMB, 64), lambda i, t: (jnp.minimum(t, 1), 0, 0)),
            pl.BlockSpec((1, 1, 64), lambda i, t: (jnp.minimum(t, 1), 0, 0)),
            pl.BlockSpec((1, 64, 32), lambda i, t: (jnp.minimum(t, 1), 0, 0)),
            pl.BlockSpec((1, 1, 32), lambda i, t: (jnp.minimum(t, 1), 0, 0)),
        ],
        out_specs=[
            pl.BlockSpec((CB, 32), lambda i, t: (i, 0)),
            pl.BlockSpec((CB, 32), lambda i, t: (i, 0)),
            pl.BlockSpec((CB, 32), lambda i, t: (i, 0)),
        ],
        out_shape=[ovec, ovec, ovec],
    )(emb3, w1s, b1s, w2s, b2s)


def kernel(user_ids, pos_movie_ids, neg_movie_ids, user_table, movie_table,
           uW1, ub1, uW2, ub2, mW1, mb1, mW2, mb2):
    ids3 = jnp.stack([user_ids, pos_movie_ids, neg_movie_ids]).astype(jnp.int32)
    ids3 = ids3.reshape(3, B // CH, CH)
    user_t3 = user_table.reshape(1000000 // SUB, SUB, EMB)
    movie_t3 = movie_table.reshape(1000000 // SUB, SUB, EMB)
    emb3 = _sc_gather(user_t3, movie_t3, ids3)
    w1s = jnp.stack([uW1, mW1])
    b1s = jnp.stack([ub1, mb1]).reshape(2, 1, 64)
    w2s = jnp.stack([uW2, mW2])
    b2s = jnp.stack([ub2, mb2]).reshape(2, 1, 32)
    return _tc_mlp(emb3, w1s, b1s, w2s, b2s)
